# baseline (speedup 1.0000x reference)
import functools

import jax
import jax.numpy as jnp
from jax import lax
from jax.experimental import pallas as pl
from jax.experimental.pallas import tpu as pltpu
from jax.experimental.pallas import tpu_sc as plsc

VOC = 1000001
DIM = 64
NC = 2
NS = 16
NW = NC * NS
CH = 128
NRB = 4
NWB = 2
LEAD = 3

PAIR_BLK = 16384
HALF_BLK = PAIR_BLK // 2
N_BLK = (VOC + PAIR_BLK - 1) // PAIR_BLK

FIX_BB = 2048


def _pair_body(x_ref, o_ref):
    t = x_ref[...].T
    o_ref[:, 0:DIM] = t[0:HALF_BLK, :]
    o_ref[:, DIM : 2 * DIM] = t[HALF_BLK:PAIR_BLK, :]


def _pair_table(emb_t):
    return pl.pallas_call(
        _pair_body,
        grid=(N_BLK,),
        in_specs=[pl.BlockSpec((DIM, PAIR_BLK), lambda i: (0, i))],
        out_specs=pl.BlockSpec((HALF_BLK, 2 * DIM), lambda i: (i, 0)),
        out_shape=jax.ShapeDtypeStruct((N_BLK * HALF_BLK, 2 * DIM),
                                       jnp.float32),
    )(emb_t)


def _fix_body(x_ref, o_ref):
    o_ref[...] = x_ref[...].T[None]


def _untranspose(out_flat, batch, hist):
    nb = batch // FIX_BB
    return pl.pallas_call(
        _fix_body,
        grid=(hist, nb),
        in_specs=[pl.BlockSpec((FIX_BB, DIM), lambda h, i: (h * nb + i, 0))],
        out_specs=pl.BlockSpec((1, DIM, FIX_BB), lambda h, i: (h, 0, i)),
        out_shape=jax.ShapeDtypeStruct((hist, DIM, batch), jnp.float32),
    )(out_flat)


def _make_gather(nbatch: int, hist: int):
    total = nbatch * hist
    assert total % (NW * CH * NRB) == 0
    assert nbatch % CH == 0 and (nbatch & (nbatch - 1)) == 0
    bshift = nbatch.bit_length() - 1
    bpw = total // NW
    nch = bpw // CH
    assert nch % NRB == 0 and nch >= 2 * NRB

    mesh = plsc.VectorSubcoreMesh(core_axis_name="c", subcore_axis_name="s")

    @functools.partial(
        pl.kernel,
        mesh=mesh,
        compiler_params=pltpu.CompilerParams(needs_layout_passes=False),
        out_type=jax.ShapeDtypeStruct((hist, DIM, nbatch), jnp.float32),
        scratch_types=[
            pltpu.VMEM((nch, CH), jnp.int32),
            pltpu.VMEM((NRB, CH), jnp.int32),
            pltpu.VMEM((NRB, CH), jnp.int32),
            pltpu.VMEM((NRB, CH, 2 * DIM), jnp.float32),
            pltpu.VMEM((NWB, DIM, CH), jnp.float32),
            pltpu.SemaphoreType.DMA,
            pltpu.SemaphoreType.DMA,
        ],
    )
    def emb(table_hbm, idx_hbm, out_hbm, idx_v, pair_v, half_v, rows_v,
            outb_v, gsem, wsem):
        wid = lax.axis_index("s") * NC + lax.axis_index("c")
        base = wid * bpw
        pltpu.sync_copy(idx_hbm.at[wid], idx_v)

        def prep_chunk(j, slot):
            for g in range(CH // 16):
                r = idx_v[j, pl.ds(g * 16, 16)]
                blk = lax.shift_right_logical(r, 14)
                pair_v[slot, pl.ds(g * 16, 16)] = (
                    lax.shift_left(blk, 13)
                    + lax.bitwise_and(r, HALF_BLK - 1))
                half_v[slot, pl.ds(g * 16, 16)] = lax.shift_left(
                    lax.bitwise_and(lax.shift_right_logical(r, 13), 1), 6)

        def start_gather(slot):
            pltpu.async_copy(
                table_hbm.at[pair_v.at[slot]], rows_v.at[slot], gsem)

        def wait_gather(slot):
            pltpu.make_async_copy(
                table_hbm.at[pair_v.at[slot]], rows_v.at[slot], gsem).wait()

        def repack(rs, os):
            @pl.loop(0, CH // 16)
            def grp(g):
                lanes0 = half_v[rs, pl.ds(g * 16, 16)]
                rows16 = g * 16 + lax.iota(jnp.int32, 16)
                src = rows_v.at[rs]
                for f in range(DIM):
                    outb_v[os, f, pl.ds(g * 16, 16)] = plsc.load_gather(
                        src, [rows16, lanes0 + f])

        def start_write(j, os):
            flat0 = base + j * CH
            h = lax.shift_right_logical(flat0, bshift)
            b0 = pl.multiple_of(lax.bitwise_and(flat0, nbatch - 1), CH)
            pltpu.async_copy(
                outb_v.at[os], out_hbm.at[h, :, pl.ds(b0, CH)], wsem)

        def wait_write(os):
            pltpu.make_async_copy(
                outb_v.at[os], out_hbm.at[0, :, pl.ds(0, CH)], wsem).wait()

        for j in range(LEAD):
            prep_chunk(j, j)
            start_gather(j)

        @pl.loop(0, nch)
        def step(j):
            rs = lax.bitwise_and(j, NRB - 1)
            os = lax.bitwise_and(j, NWB - 1)
            wait_gather(rs)

            @pl.when(j >= NWB)
            def _():
                wait_write(os)

            repack(rs, os)
            start_write(j, os)

            @pl.when(j + LEAD < nch)
            def _():
                prep_chunk(j + LEAD, lax.bitwise_and(j + LEAD, NRB - 1))
                start_gather(lax.bitwise_and(j + LEAD, NRB - 1))

        for j in range(nch - NWB, nch):
            wait_write(j % NWB)

    return emb


def kernel(x, embeddings):
    batch, hist = x.shape
    total = batch * hist
    table2 = _pair_table(embeddings.T)
    idx = x.T.reshape(NW, total // (NW * CH), CH)
    out3 = _make_gather(batch, hist)(table2, idx)
    return out3.transpose(2, 0, 1)


# speedup vs baseline: 1.6167x; 1.6167x over previous
import functools

import jax
import jax.numpy as jnp
from jax import lax
from jax.experimental import pallas as pl
from jax.experimental.pallas import tpu as pltpu
from jax.experimental.pallas import tpu_sc as plsc

VOC = 1000001
DIM = 64
NC = 2
NS = 16
NW = NC * NS
CH = 128
NRB = 4
NWB = 2
LEAD = 3

PAIR_BLK = 16384
HALF_BLK = PAIR_BLK // 2
N_BLK = (VOC + PAIR_BLK - 1) // PAIR_BLK

FIX_BB = 2048


def _pair_body(x_ref, o_ref):
    t = x_ref[...].T
    o_ref[:, 0:DIM] = t[0:HALF_BLK, :]
    o_ref[:, DIM : 2 * DIM] = t[HALF_BLK:PAIR_BLK, :]


def _pair_table(emb_t):
    return pl.pallas_call(
        _pair_body,
        grid=(N_BLK,),
        in_specs=[pl.BlockSpec((DIM, PAIR_BLK), lambda i: (0, i))],
        out_specs=pl.BlockSpec((HALF_BLK, 2 * DIM), lambda i: (i, 0)),
        out_shape=jax.ShapeDtypeStruct((N_BLK * HALF_BLK, 2 * DIM),
                                       jnp.float32),
    )(emb_t)


def _fix_body(x_ref, o_ref):
    o_ref[...] = x_ref[...].T[None]


def _untranspose(out_flat, batch, hist):
    nb = batch // FIX_BB
    return pl.pallas_call(
        _fix_body,
        grid=(hist, nb),
        in_specs=[pl.BlockSpec((FIX_BB, DIM), lambda h, i: (h * nb + i, 0))],
        out_specs=pl.BlockSpec((1, DIM, FIX_BB), lambda h, i: (h, 0, i)),
        out_shape=jax.ShapeDtypeStruct((hist, DIM, batch), jnp.float32),
    )(out_flat)


def _make_gather(nbatch: int, hist: int):
    total = nbatch * hist
    assert total % (NW * CH * NRB) == 0
    assert nbatch % CH == 0 and (nbatch & (nbatch - 1)) == 0
    bshift = nbatch.bit_length() - 1
    bpw = total // NW
    nch = bpw // CH
    assert nch % NRB == 0 and nch >= 2 * NRB

    mesh = plsc.VectorSubcoreMesh(core_axis_name="c", subcore_axis_name="s")

    @functools.partial(
        pl.kernel,
        mesh=mesh,
        out_type=jax.ShapeDtypeStruct((total, DIM), jnp.float32),
        scratch_types=[
            pltpu.VMEM((nch, CH), jnp.int32),
            pltpu.VMEM((NRB, CH), jnp.int32),
            pltpu.VMEM((NRB, CH), jnp.int32),
            pltpu.VMEM((NRB, CH, 2 * DIM), jnp.float32),
            pltpu.VMEM((NWB, CH, DIM), jnp.float32),
            pltpu.SemaphoreType.DMA,
            pltpu.SemaphoreType.DMA,
        ],
    )
    def emb(table_hbm, idx_hbm, out_hbm, idx_v, pair_v, half_v, rows_v,
            outb_v, gsem, wsem):
        wid = lax.axis_index("s") * NC + lax.axis_index("c")
        base = wid * bpw
        pltpu.sync_copy(idx_hbm.at[wid], idx_v)

        def prep_chunk(j, slot):
            for g in range(CH // 16):
                r = idx_v[j, pl.ds(g * 16, 16)]
                blk = lax.shift_right_logical(r, 14)
                pair_v[slot, pl.ds(g * 16, 16)] = (
                    lax.shift_left(blk, 13)
                    + lax.bitwise_and(r, HALF_BLK - 1))
                half_v[slot, pl.ds(g * 16, 16)] = lax.shift_left(
                    lax.bitwise_and(lax.shift_right_logical(r, 13), 1), 6)

        def start_gather(slot):
            pltpu.async_copy(
                table_hbm.at[pair_v.at[slot]], rows_v.at[slot], gsem)

        def wait_gather(slot):
            pltpu.make_async_copy(
                table_hbm.at[pair_v.at[slot]], rows_v.at[slot], gsem).wait()

        def repack(rs, os):
            @pl.loop(0, CH // 16)
            def grp(g):
                hv = half_v[rs, pl.ds(g * 16, 16)]
                for e in range(16):
                    i = g * 16 + e
                    off = hv[e]
                    for k in range(DIM // 16):
                        outb_v[os, i, pl.ds(k * 16, 16)] = (
                            rows_v[rs, i, pl.ds(off + k * 16, 16)])

        def start_write(j, os):
            pltpu.async_copy(
                outb_v.at[os], out_hbm.at[pl.ds(base + j * CH, CH)], wsem)

        def wait_write(os):
            pltpu.make_async_copy(
                outb_v.at[os], out_hbm.at[pl.ds(base, CH)], wsem).wait()

        for j in range(LEAD):
            prep_chunk(j, j)
            start_gather(j)

        @pl.loop(0, nch)
        def step(j):
            rs = lax.bitwise_and(j, NRB - 1)
            os = lax.bitwise_and(j, NWB - 1)
            wait_gather(rs)

            @pl.when(j >= NWB)
            def _():
                wait_write(os)

            repack(rs, os)
            start_write(j, os)

            @pl.when(j + LEAD < nch)
            def _():
                prep_chunk(j + LEAD, lax.bitwise_and(j + LEAD, NRB - 1))
                start_gather(lax.bitwise_and(j + LEAD, NRB - 1))

        for j in range(nch - NWB, nch):
            wait_write(j % NWB)

    return emb


def kernel(x, embeddings):
    batch, hist = x.shape
    total = batch * hist
    table2 = _pair_table(embeddings.T)
    idx = x.T.reshape(NW, total // (NW * CH), CH)
    out = _make_gather(batch, hist)(table2, idx)
    return out.reshape(hist, batch, DIM).transpose(1, 0, 2)
